# baseline (speedup 1.0000x reference)
import functools

import jax
import jax.numpy as jnp
from jax import lax
from jax.experimental import pallas as pl
from jax.experimental.pallas import tpu as pltpu
from jax.experimental.pallas import tpu_sc as plsc

N = 10000
NP = 10240
D = 128
H1 = 32
H2 = 16
NC = 2
NS = 16
TILES = NC * NS
B = 128
E = 320000
EPT = E // TILES
KF = EPT // B
TAIL = EPT - KF * B
EROWS = 2560
RW = EROWS // TILES
RPT = NP // NS

_mesh = plsc.VectorSubcoreMesh(core_axis_name="c", subcore_axis_name="s")


def _hist_body(dst_hbm, out_hbm, dst_v, ones_v, zer_v, deg_sh, sem):
    cid = lax.axis_index("c")
    tid = lax.axis_index("s")
    w = cid * NS + tid
    r0 = tid * RPT
    nwin = RW
    pltpu.sync_copy(dst_hbm.at[pl.ds(w * RW, RW)], dst_v)
    for i in range(B // 16):
        ones_v[pl.ds(i * 16, 16)] = jnp.ones((16,), jnp.float32)
    for i in range(RPT // 16):
        zer_v[pl.ds(i * 16, 16)] = jnp.zeros((16,), jnp.float32)
    pltpu.sync_copy(zer_v, deg_sh.at[pl.ds(r0, RPT)])
    plsc.subcore_barrier()

    @pl.loop(0, nwin)
    def _(k):
        pltpu.async_copy(ones_v, deg_sh.at[dst_v.at[k]], sem, add=True)

    @pl.loop(0, nwin)
    def _(k):
        pltpu.make_async_copy(ones_v, deg_sh.at[dst_v.at[k]], sem).wait()

    plsc.subcore_barrier()
    pltpu.sync_copy(deg_sh.at[pl.ds(r0, RPT)], out_hbm.at[cid, pl.ds(r0, RPT)])


def _sc_hist(dst2):
    return pl.kernel(
        _hist_body,
        out_type=jax.ShapeDtypeStruct((NC, NP), jnp.float32),
        mesh=_mesh,
        scratch_types=[
            pltpu.VMEM((RW, B), jnp.int32),
            pltpu.VMEM((B,), jnp.float32),
            pltpu.VMEM((RPT,), jnp.float32),
            pltpu.VMEM_SHARED((NP,), jnp.float32),
            pltpu.SemaphoreType.DMA,
        ],
    )(dst2)


def _scat_body(F, g_hbm, src_hbm, dst_hbm, out_hbm,
               src_v, dwin, dwin_t, rows, rows_t, zbuf, g_sh, acc_sh,
               gsems, dsems):
    cid = lax.axis_index("c")
    tid = lax.axis_index("s")
    w = cid * NS + tid
    r0 = tid * RPT
    e0 = w * EPT
    pltpu.sync_copy(src_hbm.at[pl.ds(e0, EPT)], src_v)
    pltpu.sync_copy(g_hbm.at[pl.ds(r0, RPT)], g_sh.at[pl.ds(r0, RPT)])
    for i in range(16):
        for c in range(F // 16):
            zbuf[i, pl.ds(c * 16, 16)] = jnp.zeros((16,), jnp.float32)
    for b in range(RPT // 16):
        pltpu.sync_copy(zbuf, acc_sh.at[pl.ds(r0 + b * 16, 16)])
    plsc.subcore_barrier()

    def fire(k, b):
        base = pl.multiple_of(e0 + k * B, 16)
        pltpu.async_copy(g_sh.at[src_v.at[pl.ds(k * B, B)]], rows.at[b],
                         gsems[b])
        pltpu.async_copy(dst_hbm.at[pl.ds(base, B)], dwin.at[b], dsems[b])

    def drain(k, b):
        base = pl.multiple_of(e0 + k * B, 16)
        pltpu.make_async_copy(g_sh.at[src_v.at[pl.ds(k * B, B)]], rows.at[b],
                              gsems[b]).wait()
        pltpu.make_async_copy(dst_hbm.at[pl.ds(base, B)], dwin.at[b],
                              dsems[b]).wait()

    fire(0, 0)

    @pl.loop(0, KF, step=2)
    def _(k):
        fire(k + 1, 1)
        drain(k, 0)
        pltpu.sync_copy(rows.at[0], acc_sh.at[dwin.at[0]], add=True)
        knext = jnp.minimum(k + 2, KF - 2)
        fire(knext, 0)
        drain(k + 1, 1)
        pltpu.sync_copy(rows.at[1], acc_sh.at[dwin.at[1]], add=True)

    drain(0, 0)
    toff = e0 + KF * B
    pltpu.sync_copy(dst_hbm.at[pl.ds(toff, TAIL)], dwin_t)
    pltpu.async_copy(g_sh.at[src_v.at[pl.ds(KF * B, TAIL)]], rows_t,
                     gsems[0])
    pltpu.make_async_copy(g_sh.at[src_v.at[pl.ds(KF * B, TAIL)]], rows_t,
                          gsems[0]).wait()
    pltpu.sync_copy(rows_t, acc_sh.at[dwin_t], add=True)

    plsc.subcore_barrier()
    pltpu.sync_copy(acc_sh.at[pl.ds(r0, RPT)],
                    out_hbm.at[cid, pl.ds(r0, RPT)])


def _sc_scatter(g, src1, dst1, F):
    return pl.kernel(
        functools.partial(_scat_body, F),
        out_type=jax.ShapeDtypeStruct((NC, NP, F), jnp.float32),
        mesh=_mesh,
        scratch_types=[
            pltpu.VMEM((EPT,), jnp.int32),
            pltpu.VMEM((2, B), jnp.int32),
            pltpu.VMEM((TAIL,), jnp.int32),
            pltpu.VMEM((2, B, F), jnp.float32),
            pltpu.VMEM((TAIL, F), jnp.float32),
            pltpu.VMEM((16, F), jnp.float32),
            pltpu.VMEM_SHARED((NP, F), jnp.float32),
            pltpu.VMEM_SHARED((NP, F), jnp.float32),
            [pltpu.SemaphoreType.DMA] * 2,
            [pltpu.SemaphoreType.DMA] * 2,
        ],
        compiler_params=pltpu.CompilerParams(use_tc_tiling_on_sc=False),
    )(g, src1, dst1)


R = 2048


def _prep_body(hist_ref, x_ref, w1_ref, g1_ref, dinv_ref):
    ones2 = jnp.ones((NC, 1), jnp.float32)
    deg = lax.dot_general(hist_ref[...], ones2,
                          (((0,), (0,)), ((), ())),
                          preferred_element_type=jnp.float32)
    dinv = lax.rsqrt(1.0 + deg)
    y1 = jnp.dot(x_ref[...], w1_ref[...],
                 preferred_element_type=jnp.float32)
    g1_ref[...] = dinv * y1
    dinv_ref[...] = jnp.broadcast_to(dinv, (R, H1))


def _tc_prep(hist, x, W1):
    return pl.pallas_call(
        _prep_body,
        grid=(NP // R,),
        in_specs=[
            pl.BlockSpec((NC, R), lambda i: (0, i)),
            pl.BlockSpec((R, D), lambda i: (i, 0)),
            pl.BlockSpec((D, H1), lambda i: (0, 0)),
        ],
        out_specs=[
            pl.BlockSpec((R, H1), lambda i: (i, 0)),
            pl.BlockSpec((R, H1), lambda i: (i, 0)),
        ],
        out_shape=[
            jax.ShapeDtypeStruct((NP, H1), jnp.float32),
            jax.ShapeDtypeStruct((NP, H1), jnp.float32),
        ],
    )(hist, x, W1)


def _mid_body(s1_ref, g1_ref, dinv_ref, w2_ref, g2_ref):
    h1 = dinv_ref[...] * (s1_ref[0] + s1_ref[1] + g1_ref[...])
    y2 = jnp.dot(h1, w2_ref[...], preferred_element_type=jnp.float32)
    g2_ref[...] = dinv_ref[:, :H2] * y2


def _tc_mid(s1, g1, dinv32, W2):
    return pl.pallas_call(
        _mid_body,
        grid=(NP // R,),
        in_specs=[
            pl.BlockSpec((NC, R, H1), lambda i: (0, i, 0)),
            pl.BlockSpec((R, H1), lambda i: (i, 0)),
            pl.BlockSpec((R, H1), lambda i: (i, 0)),
            pl.BlockSpec((H1, H2), lambda i: (0, 0)),
        ],
        out_specs=pl.BlockSpec((R, H2), lambda i: (i, 0)),
        out_shape=jax.ShapeDtypeStruct((NP, H2), jnp.float32),
    )(s1, g1, dinv32, W2)


def _out_body(s2_ref, g2_ref, dinv_ref, w3_ref, w4_ref, o_ref):
    h2 = dinv_ref[:, :H2] * (s2_ref[0] + s2_ref[1] + g2_ref[...])
    w34 = jnp.dot(w3_ref[...], w4_ref[...],
                  preferred_element_type=jnp.float32)
    o_ref[...] = jnp.dot(h2, w34, preferred_element_type=jnp.float32)


def _tc_out(s2, g2, dinv32, W3, W4):
    return pl.pallas_call(
        _out_body,
        grid=(NP // R,),
        in_specs=[
            pl.BlockSpec((NC, R, H2), lambda i: (0, i, 0)),
            pl.BlockSpec((R, H2), lambda i: (i, 0)),
            pl.BlockSpec((R, H1), lambda i: (i, 0)),
            pl.BlockSpec((H2, H1), lambda i: (0, 0)),
            pl.BlockSpec((H1, D), lambda i: (0, 0)),
        ],
        out_specs=pl.BlockSpec((R, D), lambda i: (i, 0)),
        out_shape=jax.ShapeDtypeStruct((N, D), jnp.float32),
    )(s2, g2, dinv32, W3, W4)


def kernel(x, edge_index, W1, W2, W3, W4):
    src1 = edge_index[0].astype(jnp.int32)
    dst1 = edge_index[1].astype(jnp.int32)
    dst2 = jnp.pad(dst1, (0, EROWS * B - E),
                   constant_values=N).reshape(EROWS, B)

    hist = _sc_hist(dst2)
    g1, dinv32 = _tc_prep(hist, x, W1)
    s1 = _sc_scatter(g1, src1, dst1, H1)
    g2 = _tc_mid(s1, g1, dinv32, W2)
    s2 = _sc_scatter(g2, src1, dst1, H2)
    return _tc_out(s2, g2, dinv32, W3, W4)


# speedup vs baseline: 1.1620x; 1.1620x over previous
import functools

import jax
import jax.numpy as jnp
from jax import lax
from jax.experimental import pallas as pl
from jax.experimental.pallas import tpu as pltpu
from jax.experimental.pallas import tpu_sc as plsc

N = 10000
NP = 10240
D = 128
H1 = 32
H2 = 16
NC = 2
NS = 16
TILES = NC * NS
B = 128
E = 320000
EPT = E // TILES
KF = EPT // B
TAIL = EPT - KF * B
EROWS = E // B
RW = EROWS // TILES
RWMAX = EROWS - RW * (TILES - 1)
RPT = NP // NS

_mesh = plsc.VectorSubcoreMesh(core_axis_name="c", subcore_axis_name="s")


def _nwin(w):
    return jnp.where(w == TILES - 1, RWMAX, RW)


def _hist_body(ei_hbm, out_hbm, dst_v, ones_v, zer_v, deg_sh, sem):
    cid = lax.axis_index("c")
    tid = lax.axis_index("s")
    w = cid * NS + tid
    r0 = tid * RPT
    nwin = _nwin(w)
    pltpu.sync_copy(ei_hbm.at[1, pl.ds(w * RW, RWMAX)], dst_v)
    for i in range(B // 16):
        ones_v[pl.ds(i * 16, 16)] = jnp.ones((16,), jnp.float32)
    for i in range(RPT // 16):
        zer_v[pl.ds(i * 16, 16)] = jnp.zeros((16,), jnp.float32)
    pltpu.sync_copy(zer_v, deg_sh.at[pl.ds(r0, RPT)])
    plsc.subcore_barrier()

    @pl.loop(0, nwin)
    def _(k):
        pltpu.async_copy(ones_v, deg_sh.at[dst_v.at[k]], sem, add=True)

    @pl.loop(0, nwin)
    def _(k):
        pltpu.make_async_copy(ones_v, deg_sh.at[dst_v.at[k]], sem).wait()

    plsc.subcore_barrier()
    pltpu.sync_copy(deg_sh.at[pl.ds(r0, RPT)], out_hbm.at[cid, pl.ds(r0, RPT)])


def _sc_hist(ei3):
    return pl.kernel(
        _hist_body,
        out_type=jax.ShapeDtypeStruct((NC, NP), jnp.float32),
        mesh=_mesh,
        scratch_types=[
            pltpu.VMEM((RWMAX, B), jnp.int32),
            pltpu.VMEM((B,), jnp.float32),
            pltpu.VMEM((RPT,), jnp.float32),
            pltpu.VMEM_SHARED((NP,), jnp.float32),
            pltpu.SemaphoreType.DMA,
        ],
        compiler_params=pltpu.CompilerParams(use_tc_tiling_on_sc=False),
    )(ei3)


def _scat_body(F, g_hbm, ei_hbm, out_hbm,
               src_v, dst_v, rows, zbuf, g_sh, acc_sh, gsems):
    cid = lax.axis_index("c")
    tid = lax.axis_index("s")
    w = cid * NS + tid
    r0 = tid * RPT
    nwin = _nwin(w)
    pltpu.sync_copy(ei_hbm.at[0, pl.ds(w * RW, RWMAX)], src_v)
    pltpu.sync_copy(ei_hbm.at[1, pl.ds(w * RW, RWMAX)], dst_v)
    pltpu.sync_copy(g_hbm.at[pl.ds(r0, RPT)], g_sh.at[pl.ds(r0, RPT)])
    for i in range(16):
        for c in range(F // 16):
            zbuf[i, pl.ds(c * 16, 16)] = jnp.zeros((16,), jnp.float32)
    for b in range(RPT // 16):
        pltpu.sync_copy(zbuf, acc_sh.at[pl.ds(r0 + b * 16, 16)])
    plsc.subcore_barrier()

    pltpu.async_copy(g_sh.at[src_v.at[0]], rows.at[0], gsems[0])

    @pl.loop(0, nwin, step=2)
    def _(k):
        pltpu.async_copy(g_sh.at[src_v.at[k + 1]], rows.at[1], gsems[1])
        pltpu.make_async_copy(g_sh.at[src_v.at[k]], rows.at[0],
                              gsems[0]).wait()
        pltpu.sync_copy(rows.at[0], acc_sh.at[dst_v.at[k]], add=True)
        knext = jnp.minimum(k + 2, nwin - 2)
        pltpu.async_copy(g_sh.at[src_v.at[knext]], rows.at[0], gsems[0])
        pltpu.make_async_copy(g_sh.at[src_v.at[k + 1]], rows.at[1],
                              gsems[1]).wait()
        pltpu.sync_copy(rows.at[1], acc_sh.at[dst_v.at[k + 1]], add=True)

    pltpu.make_async_copy(g_sh.at[src_v.at[0]], rows.at[0], gsems[0]).wait()
    plsc.subcore_barrier()
    pltpu.sync_copy(acc_sh.at[pl.ds(r0, RPT)],
                    out_hbm.at[cid, pl.ds(r0, RPT)])


def _sc_scatter(g, ei3, F):
    return pl.kernel(
        functools.partial(_scat_body, F),
        out_type=jax.ShapeDtypeStruct((NC, NP, F), jnp.float32),
        mesh=_mesh,
        scratch_types=[
            pltpu.VMEM((RWMAX, B), jnp.int32),
            pltpu.VMEM((RWMAX, B), jnp.int32),
            pltpu.VMEM((2, B, F), jnp.float32),
            pltpu.VMEM((16, F), jnp.float32),
            pltpu.VMEM_SHARED((NP, F), jnp.float32),
            pltpu.VMEM_SHARED((NP, F), jnp.float32),
            [pltpu.SemaphoreType.DMA] * 2,
        ],
        compiler_params=pltpu.CompilerParams(use_tc_tiling_on_sc=False),
    )(g, ei3)


R = 2048


def _prep_body(hist_ref, x_ref, w1_ref, g1_ref, dinv_ref):
    ones2 = jnp.ones((NC, 1), jnp.float32)
    deg = lax.dot_general(hist_ref[...], ones2,
                          (((0,), (0,)), ((), ())),
                          preferred_element_type=jnp.float32)
    dinv = lax.rsqrt(1.0 + deg)
    y1 = jnp.dot(x_ref[...], w1_ref[...],
                 preferred_element_type=jnp.float32)
    g1_ref[...] = dinv * y1
    dinv_ref[...] = jnp.broadcast_to(dinv, (R, H1))


def _tc_prep(hist, x, W1):
    return pl.pallas_call(
        _prep_body,
        grid=(NP // R,),
        in_specs=[
            pl.BlockSpec((NC, R), lambda i: (0, i)),
            pl.BlockSpec((R, D), lambda i: (i, 0)),
            pl.BlockSpec((D, H1), lambda i: (0, 0)),
        ],
        out_specs=[
            pl.BlockSpec((R, H1), lambda i: (i, 0)),
            pl.BlockSpec((R, H1), lambda i: (i, 0)),
        ],
        out_shape=[
            jax.ShapeDtypeStruct((NP, H1), jnp.float32),
            jax.ShapeDtypeStruct((NP, H1), jnp.float32),
        ],
    )(hist, x, W1)


def _mid_body(s1_ref, g1_ref, dinv_ref, w2_ref, g2_ref):
    h1 = dinv_ref[...] * (s1_ref[0] + s1_ref[1] + g1_ref[...])
    y2 = jnp.dot(h1, w2_ref[...], preferred_element_type=jnp.float32)
    g2_ref[...] = dinv_ref[:, :H2] * y2


def _tc_mid(s1, g1, dinv32, W2):
    return pl.pallas_call(
        _mid_body,
        grid=(NP // R,),
        in_specs=[
            pl.BlockSpec((NC, R, H1), lambda i: (0, i, 0)),
            pl.BlockSpec((R, H1), lambda i: (i, 0)),
            pl.BlockSpec((R, H1), lambda i: (i, 0)),
            pl.BlockSpec((H1, H2), lambda i: (0, 0)),
        ],
        out_specs=pl.BlockSpec((R, H2), lambda i: (i, 0)),
        out_shape=jax.ShapeDtypeStruct((NP, H2), jnp.float32),
    )(s1, g1, dinv32, W2)


def _out_body(s2_ref, g2_ref, dinv_ref, w3_ref, w4_ref, o_ref):
    h2 = dinv_ref[:, :H2] * (s2_ref[0] + s2_ref[1] + g2_ref[...])
    w34 = jnp.dot(w3_ref[...], w4_ref[...],
                  preferred_element_type=jnp.float32)
    o_ref[...] = jnp.dot(h2, w34, preferred_element_type=jnp.float32)


def _tc_out(s2, g2, dinv32, W3, W4):
    return pl.pallas_call(
        _out_body,
        grid=(NP // R,),
        in_specs=[
            pl.BlockSpec((NC, R, H2), lambda i: (0, i, 0)),
            pl.BlockSpec((R, H2), lambda i: (i, 0)),
            pl.BlockSpec((R, H1), lambda i: (i, 0)),
            pl.BlockSpec((H2, H1), lambda i: (0, 0)),
            pl.BlockSpec((H1, D), lambda i: (0, 0)),
        ],
        out_specs=pl.BlockSpec((R, D), lambda i: (i, 0)),
        out_shape=jax.ShapeDtypeStruct((N, D), jnp.float32),
    )(s2, g2, dinv32, W3, W4)


def kernel(x, edge_index, W1, W2, W3, W4):
    ei3 = edge_index.astype(jnp.int32).reshape(2, EROWS, B)

    hist = _sc_hist(ei3)
    g1, dinv32 = _tc_prep(hist, x, W1)
    s1 = _sc_scatter(g1, ei3, H1)
    g2 = _tc_mid(s1, g1, dinv32, W2)
    s2 = _sc_scatter(g2, ei3, H2)
    return _tc_out(s2, g2, dinv32, W3, W4)
